# baseline (speedup 1.0000x reference)
import functools

import jax
import jax.numpy as jnp
from jax.experimental import pallas as pl
from jax.experimental.pallas import tpu as pltpu

N = 10000
F = 128
BM = 200
BM2 = 1000
EPS = 1e-5


def _mm_kernel(adj_ref, seq_ref, w1t_ref, h_ref, stats_ref):
    p = jnp.dot(adj_ref[...], seq_ref[...],
                preferred_element_type=jnp.float32)
    h = jnp.dot(p, w1t_ref[...], preferred_element_type=jnp.float32)
    h_ref[...] = h

    @pl.when(pl.program_id(0) == 0)
    def _zero_stats():
        stats_ref[...] = jnp.zeros_like(stats_ref)

    stats_ref[0:1, :] += jnp.sum(h, axis=0, keepdims=True)
    stats_ref[1:2, :] += jnp.sum(h * h, axis=0, keepdims=True)


def _bn_kernel(h_ref, self_ref, w2at_ref, w2bt_ref, stats_ref, gb_ref,
               out_ref):
    inv_n = 1.0 / N
    mean = stats_ref[0:1, :] * inv_n
    var = stats_ref[1:2, :] * inv_n - mean * mean
    scale = gb_ref[0:1, :] * jax.lax.rsqrt(var + EPS)
    shift = gb_ref[1:2, :] - mean * scale
    t = jnp.tanh(h_ref[...] * scale + shift)
    out = jnp.dot(self_ref[...], w2at_ref[...],
                  preferred_element_type=jnp.float32)
    out += jnp.dot(t, w2bt_ref[...], preferred_element_type=jnp.float32)
    out_ref[...] = jnp.tanh(out)


@functools.partial(jax.jit, static_argnames=())
def kernel(seq_self, seq, adj, W1, W2, gamma, beta):
    w1t = W1.T
    w2at = W2[:, :F].T
    w2bt = W2[:, F:].T
    gb = jnp.zeros((8, F), jnp.float32).at[0].set(gamma).at[1].set(beta)

    h, stats = pl.pallas_call(
        _mm_kernel,
        grid=(N // BM,),
        in_specs=[
            pl.BlockSpec((BM, N), lambda i: (i, 0)),
            pl.BlockSpec((N, F), lambda i: (0, 0)),
            pl.BlockSpec((F, F), lambda i: (0, 0)),
        ],
        out_specs=[
            pl.BlockSpec((BM, F), lambda i: (i, 0)),
            pl.BlockSpec((8, F), lambda i: (0, 0)),
        ],
        out_shape=[
            jax.ShapeDtypeStruct((N, F), jnp.float32),
            jax.ShapeDtypeStruct((8, F), jnp.float32),
        ],
        compiler_params=pltpu.CompilerParams(
            dimension_semantics=("arbitrary",)),
    )(adj, seq, w1t)

    out = pl.pallas_call(
        _bn_kernel,
        grid=(N // BM2,),
        in_specs=[
            pl.BlockSpec((BM2, F), lambda i: (i, 0)),
            pl.BlockSpec((BM2, F), lambda i: (i, 0)),
            pl.BlockSpec((F, F), lambda i: (0, 0)),
            pl.BlockSpec((F, F), lambda i: (0, 0)),
            pl.BlockSpec((8, F), lambda i: (0, 0)),
            pl.BlockSpec((8, F), lambda i: (0, 0)),
        ],
        out_specs=pl.BlockSpec((BM2, F), lambda i: (i, 0)),
        out_shape=jax.ShapeDtypeStruct((N, F), jnp.float32),
        compiler_params=pltpu.CompilerParams(
            dimension_semantics=("arbitrary",)),
    )(h, seq_self, w2at, w2bt, stats, gb)
    return out
